# baseline (speedup 1.0000x reference)
import functools

import jax
import jax.numpy as jnp
import numpy as np
from jax import lax
from jax.experimental import pallas as pl
from jax.experimental.pallas import tpu as pltpu

_N = 20000
_ROWS = 160
_NPAD = _ROWS * 128
_PRE = 2000
_PRE_ROWS = 16
_PREPAD = _PRE_ROWS * 128
_POST = 1000
_OUT_ROWS = 8
_NMS_T = 0.7
_IMG_W = 1024.0
_IMG_H = 1024.0
_BBOX_CLIP = float(np.log(1000.0 / 16.0))
_NEG = float("-inf")


def _rpn_kernel(a0r, a1r, a2r, a3r, d0r, d1r, d2r, d3r, scr,
                o1r, o2r, o3r, o4r,
                x1s, y1s, x2s, y2s, ssr):
    a0 = a0r[...]
    a1 = a1r[...]
    a2 = a2r[...]
    a3 = a3r[...]
    widths = a2 - a0 + 1.0
    heights = a3 - a1 + 1.0
    ctr_x = a0 + 0.5 * widths
    ctr_y = a1 + 0.5 * heights
    dw = jnp.minimum(d2r[...], _BBOX_CLIP)
    dh = jnp.minimum(d3r[...], _BBOX_CLIP)
    pcx = d0r[...] * widths + ctr_x
    pcy = d1r[...] * heights + ctr_y
    pw = jnp.exp(dw) * widths
    ph = jnp.exp(dh) * heights
    x1s[...] = jnp.clip(pcx - 0.5 * pw, 0.0, _IMG_W - 1.0)
    y1s[...] = jnp.clip(pcy - 0.5 * ph, 0.0, _IMG_H - 1.0)
    x2s[...] = jnp.clip(pcx + 0.5 * pw - 1.0, 0.0, _IMG_W - 1.0)
    y2s[...] = jnp.clip(pcy + 0.5 * ph - 1.0, 0.0, _IMG_H - 1.0)
    ssr[...] = scr[...]

    flat_big = (lax.broadcasted_iota(jnp.int32, (_ROWS, 128), 0) * 128
                + lax.broadcasted_iota(jnp.int32, (_ROWS, 128), 1))
    flat_sm = (lax.broadcasted_iota(jnp.int32, (_PRE_ROWS, 128), 0) * 128
               + lax.broadcasted_iota(jnp.int32, (_PRE_ROWS, 128), 1))
    flat_out = (lax.broadcasted_iota(jnp.int32, (_OUT_ROWS, 128), 0) * 128
                + lax.broadcasted_iota(jnp.int32, (_OUT_ROWS, 128), 1))

    flat8 = (lax.broadcasted_iota(jnp.int32, (8, 128), 0) * 128
             + lax.broadcasted_iota(jnp.int32, (8, 128), 1))
    lane1 = lax.broadcasted_iota(jnp.int32, (1, 128), 1)

    def init_rm(r, rm):
        row = ssr[pl.ds(r, 1), :]
        return jnp.where(flat8 == r, jnp.max(row), rm)

    rm0 = lax.fori_loop(0, _ROWS, init_rm,
                        jnp.full((8, 128), _NEG, jnp.float32))

    def p1(j, carry):
        rm, bx1, by1, bx2, by2, bs = carry
        m = jnp.max(rm)
        r = jnp.min(jnp.where(rm == m, flat8, _ROWS))
        row = ssr[pl.ds(r, 1), :]
        c = jnp.min(jnp.where(row == m, lane1, 128))
        oh = lane1 == c
        gx1 = jnp.sum(jnp.where(oh, x1s[pl.ds(r, 1), :], 0.0))
        gy1 = jnp.sum(jnp.where(oh, y1s[pl.ds(r, 1), :], 0.0))
        gx2 = jnp.sum(jnp.where(oh, x2s[pl.ds(r, 1), :], 0.0))
        gy2 = jnp.sum(jnp.where(oh, y2s[pl.ds(r, 1), :], 0.0))
        newrow = jnp.where(oh, _NEG, row)
        ssr[pl.ds(r, 1), :] = newrow
        rm = jnp.where(flat8 == r, jnp.max(newrow), rm)
        tgt = flat_sm == j
        bx1 = jnp.where(tgt, gx1, bx1)
        by1 = jnp.where(tgt, gy1, by1)
        bx2 = jnp.where(tgt, gx2, bx2)
        by2 = jnp.where(tgt, gy2, by2)
        bs = jnp.where(tgt, m, bs)
        return (rm, bx1, by1, bx2, by2, bs)

    zeros_sm = jnp.zeros((_PRE_ROWS, 128), jnp.float32)
    neg_sm = jnp.full((_PRE_ROWS, 128), _NEG, jnp.float32)
    _, bx1, by1, bx2, by2, bs = lax.fori_loop(
        0, _PRE, p1, (rm0, zeros_sm, zeros_sm, zeros_sm, zeros_sm, neg_sm))

    areas = (bx2 - bx1 + 1.0) * (by2 - by1 + 1.0)
    keep0 = jnp.where(flat_sm < _PRE, 1.0, 0.0)

    def p2(i, keep):
        oh = flat_sm == i
        xi1 = jnp.sum(jnp.where(oh, bx1, 0.0))
        yi1 = jnp.sum(jnp.where(oh, by1, 0.0))
        xi2 = jnp.sum(jnp.where(oh, bx2, 0.0))
        yi2 = jnp.sum(jnp.where(oh, by2, 0.0))
        ai = jnp.sum(jnp.where(oh, areas, 0.0))
        ki = jnp.sum(jnp.where(oh, keep, 0.0))
        xx1 = jnp.maximum(xi1, bx1)
        yy1 = jnp.maximum(yi1, by1)
        xx2 = jnp.minimum(xi2, bx2)
        yy2 = jnp.minimum(yi2, by2)
        w = jnp.maximum(xx2 - xx1 + 1.0, 0.0)
        h = jnp.maximum(yy2 - yy1 + 1.0, 0.0)
        inter = w * h
        iou = inter / (ai + areas - inter)
        supp = (iou > _NMS_T) & (flat_sm > i) & (ki > 0.5)
        return jnp.where(supp, 0.0, keep)

    keep = lax.fori_loop(0, _PRE, p2, keep0)

    masked0 = jnp.where(keep > 0.5, bs, _NEG)
    zeros_out = jnp.zeros((_OUT_ROWS, 128), jnp.float32)

    def p3(o, carry):
        m1, m2, m3, m4, msk = carry
        mm = jnp.max(msk)
        sel = jnp.min(jnp.where(msk == mm, flat_sm, _PREPAD))
        oh = flat_sm == sel
        g1 = jnp.sum(jnp.where(oh, bx1, 0.0))
        g2 = jnp.sum(jnp.where(oh, by1, 0.0))
        g3 = jnp.sum(jnp.where(oh, bx2, 0.0))
        g4 = jnp.sum(jnp.where(oh, by2, 0.0))
        msk = jnp.where(oh, _NEG, msk)
        tgt = flat_out == o
        m1 = jnp.where(tgt, g1, m1)
        m2 = jnp.where(tgt, g2, m2)
        m3 = jnp.where(tgt, g3, m3)
        m4 = jnp.where(tgt, g4, m4)
        return (m1, m2, m3, m4, msk)

    m1, m2, m3, m4, _ = lax.fori_loop(
        0, _POST, p3, (zeros_out, zeros_out, zeros_out, zeros_out, masked0))
    o1r[...] = m1
    o2r[...] = m2
    o3r[...] = m3
    o4r[...] = m4


@jax.jit
def kernel(anchors, objectness, pred_bbox_deltas):
    pad = _NPAD - _N
    s = jnp.pad(objectness.reshape(-1), (0, pad),
                constant_values=_NEG).reshape(_ROWS, 128)
    a = jnp.pad(anchors, ((0, pad), (0, 0)))
    d = jnp.pad(pred_bbox_deltas, ((0, pad), (0, 0)))
    cols = [a[:, i].reshape(_ROWS, 128) for i in range(4)]
    dcols = [d[:, i].reshape(_ROWS, 128) for i in range(4)]
    out_shape = [jax.ShapeDtypeStruct((_OUT_ROWS, 128), jnp.float32)] * 4
    scratch = [pltpu.VMEM((_ROWS, 128), jnp.float32)] * 5
    o1, o2, o3, o4 = pl.pallas_call(
        _rpn_kernel,
        out_shape=out_shape,
        scratch_shapes=scratch,
    )(*cols, *dcols, s)
    return jnp.stack([o.reshape(-1)[:_POST] for o in (o1, o2, o3, o4)],
                     axis=1)


# speedup vs baseline: 1.6039x; 1.6039x over previous
import jax
import jax.numpy as jnp
import numpy as np
from jax import lax
from jax.experimental import pallas as pl
from jax.experimental.pallas import tpu as pltpu

_N = 20000
_ROWS = 160
_NPAD = _ROWS * 128
_PRE = 2000
_PRE_ROWS = 16
_POST = 1000
_OUT_ROWS = 8
_NMS_T = 0.7
_IMG_W = 1024.0
_IMG_H = 1024.0
_BBOX_CLIP = float(np.log(1000.0 / 16.0))
_NEG = float("-inf")


def _rpn_kernel(a0r, a1r, a2r, a3r, d0r, d1r, d2r, d3r, scr,
                o1r, o2r, o3r, o4r,
                x1s, y1s, x2s, y2s, ssr):
    a0 = a0r[...]
    a1 = a1r[...]
    a2 = a2r[...]
    a3 = a3r[...]
    widths = a2 - a0 + 1.0
    heights = a3 - a1 + 1.0
    ctr_x = a0 + 0.5 * widths
    ctr_y = a1 + 0.5 * heights
    dw = jnp.minimum(d2r[...], _BBOX_CLIP)
    dh = jnp.minimum(d3r[...], _BBOX_CLIP)
    pcx = d0r[...] * widths + ctr_x
    pcy = d1r[...] * heights + ctr_y
    pw = jnp.exp(dw) * widths
    ph = jnp.exp(dh) * heights
    x1s[...] = jnp.clip(pcx - 0.5 * pw, 0.0, _IMG_W - 1.0)
    y1s[...] = jnp.clip(pcy - 0.5 * ph, 0.0, _IMG_H - 1.0)
    x2s[...] = jnp.clip(pcx + 0.5 * pw - 1.0, 0.0, _IMG_W - 1.0)
    y2s[...] = jnp.clip(pcy + 0.5 * ph - 1.0, 0.0, _IMG_H - 1.0)
    ssr[...] = scr[...]

    flat_big = (lax.broadcasted_iota(jnp.int32, (_ROWS, 128), 0) * 128
                + lax.broadcasted_iota(jnp.int32, (_ROWS, 128), 1))
    flat_sm = (lax.broadcasted_iota(jnp.int32, (_PRE_ROWS, 128), 0) * 128
               + lax.broadcasted_iota(jnp.int32, (_PRE_ROWS, 128), 1))
    flat_out = (lax.broadcasted_iota(jnp.int32, (_OUT_ROWS, 128), 0) * 128
                + lax.broadcasted_iota(jnp.int32, (_OUT_ROWS, 128), 1))

    zeros_sm = jnp.zeros((_PRE_ROWS, 128), jnp.float32)

    def body(j, carry):
        kx1, ky1, kx2, ky2, ka, sx1, sy1, sx2, sy2, kj, sj = carry
        s = ssr[...]
        m = jnp.max(s)
        sel = jnp.min(jnp.where(s == m, flat_big, _NPAD))
        oh = flat_big == sel
        gx1 = jnp.sum(jnp.where(oh, x1s[...], 0.0))
        gy1 = jnp.sum(jnp.where(oh, y1s[...], 0.0))
        gx2 = jnp.sum(jnp.where(oh, x2s[...], 0.0))
        gy2 = jnp.sum(jnp.where(oh, y2s[...], 0.0))
        ssr[...] = jnp.where(oh, _NEG, s)
        aj = (gx2 - gx1 + 1.0) * (gy2 - gy1 + 1.0)
        xx1 = jnp.maximum(gx1, kx1)
        yy1 = jnp.maximum(gy1, ky1)
        xx2 = jnp.minimum(gx2, kx2)
        yy2 = jnp.minimum(gy2, ky2)
        w = jnp.maximum(xx2 - xx1 + 1.0, 0.0)
        h = jnp.maximum(yy2 - yy1 + 1.0, 0.0)
        inter = w * h
        iou = inter / (aj + ka - inter)
        viol = jnp.max(jnp.where(iou > _NMS_T, 1.0, 0.0))
        keptj = viol < 0.5
        tgtk = (flat_sm == kj) & keptj
        kx1 = jnp.where(tgtk, gx1, kx1)
        ky1 = jnp.where(tgtk, gy1, ky1)
        kx2 = jnp.where(tgtk, gx2, kx2)
        ky2 = jnp.where(tgtk, gy2, ky2)
        ka = jnp.where(tgtk, aj, ka)
        tgts = (flat_sm == sj) & (~keptj)
        sx1 = jnp.where(tgts, gx1, sx1)
        sy1 = jnp.where(tgts, gy1, sy1)
        sx2 = jnp.where(tgts, gx2, sx2)
        sy2 = jnp.where(tgts, gy2, sy2)
        one = jnp.where(keptj, 1, 0)
        return (kx1, ky1, kx2, ky2, ka, sx1, sy1, sx2, sy2,
                kj + one, sj + (1 - one))

    init = (zeros_sm, zeros_sm,
            jnp.full((_PRE_ROWS, 128), -2.0, jnp.float32),
            jnp.full((_PRE_ROWS, 128), -2.0, jnp.float32),
            jnp.full((_PRE_ROWS, 128), 1.0, jnp.float32),
            zeros_sm, zeros_sm, zeros_sm, zeros_sm,
            jnp.int32(0), jnp.int32(0))
    (kx1, ky1, kx2, ky2, _ka,
     sx1, sy1, sx2, sy2, kj, _sj) = lax.fori_loop(0, _PRE, body, init)

    o1r[...] = kx1[0:_OUT_ROWS, :]
    o2r[...] = ky1[0:_OUT_ROWS, :]
    o3r[...] = kx2[0:_OUT_ROWS, :]
    o4r[...] = ky2[0:_OUT_ROWS, :]

    nfill = jnp.maximum(0, _POST - kj)

    def fill(t, _):
        ohs = flat_sm == t
        f1 = jnp.sum(jnp.where(ohs, sx1, 0.0))
        f2 = jnp.sum(jnp.where(ohs, sy1, 0.0))
        f3 = jnp.sum(jnp.where(ohs, sx2, 0.0))
        f4 = jnp.sum(jnp.where(ohs, sy2, 0.0))
        tgt = flat_out == (kj + t)
        o1r[...] = jnp.where(tgt, f1, o1r[...])
        o2r[...] = jnp.where(tgt, f2, o2r[...])
        o3r[...] = jnp.where(tgt, f3, o3r[...])
        o4r[...] = jnp.where(tgt, f4, o4r[...])
        return 0

    lax.fori_loop(0, nfill, fill, 0)


@jax.jit
def kernel(anchors, objectness, pred_bbox_deltas):
    pad = _NPAD - _N
    s = jnp.pad(objectness.reshape(-1), (0, pad),
                constant_values=_NEG).reshape(_ROWS, 128)
    a = jnp.pad(anchors, ((0, pad), (0, 0)))
    d = jnp.pad(pred_bbox_deltas, ((0, pad), (0, 0)))
    cols = [a[:, i].reshape(_ROWS, 128) for i in range(4)]
    dcols = [d[:, i].reshape(_ROWS, 128) for i in range(4)]
    out_shape = [jax.ShapeDtypeStruct((_OUT_ROWS, 128), jnp.float32)] * 4
    scratch = [pltpu.VMEM((_ROWS, 128), jnp.float32)] * 5
    o1, o2, o3, o4 = pl.pallas_call(
        _rpn_kernel,
        out_shape=out_shape,
        scratch_shapes=scratch,
    )(*cols, *dcols, s)
    return jnp.stack([o.reshape(-1)[:_POST] for o in (o1, o2, o3, o4)],
                     axis=1)


# speedup vs baseline: 1.9997x; 1.2468x over previous
import jax
import jax.numpy as jnp
import numpy as np
from jax import lax
from jax.experimental import pallas as pl
from jax.experimental.pallas import tpu as pltpu

_N = 20000
_ROWS = 160
_NPAD = _ROWS * 128
_PRE = 2000
_PRE_ROWS = 16
_POST = 1000
_OUT_ROWS = 8
_NMS_T = 0.7
_IMG_W = 1024.0
_IMG_H = 1024.0
_BBOX_CLIP = float(np.log(1000.0 / 16.0))
_NEG = float("-inf")


def _rpn_kernel(a0r, a1r, a2r, a3r, d0r, d1r, d2r, d3r, scr,
                o1r, o2r, o3r, o4r,
                x1s, y1s, x2s, y2s, ssr):
    a0 = a0r[...]
    a1 = a1r[...]
    a2 = a2r[...]
    a3 = a3r[...]
    widths = a2 - a0 + 1.0
    heights = a3 - a1 + 1.0
    ctr_x = a0 + 0.5 * widths
    ctr_y = a1 + 0.5 * heights
    dw = jnp.minimum(d2r[...], _BBOX_CLIP)
    dh = jnp.minimum(d3r[...], _BBOX_CLIP)
    pcx = d0r[...] * widths + ctr_x
    pcy = d1r[...] * heights + ctr_y
    pw = jnp.exp(dw) * widths
    ph = jnp.exp(dh) * heights
    x1s[...] = jnp.clip(pcx - 0.5 * pw, 0.0, _IMG_W - 1.0)
    y1s[...] = jnp.clip(pcy - 0.5 * ph, 0.0, _IMG_H - 1.0)
    x2s[...] = jnp.clip(pcx + 0.5 * pw - 1.0, 0.0, _IMG_W - 1.0)
    y2s[...] = jnp.clip(pcy + 0.5 * ph - 1.0, 0.0, _IMG_H - 1.0)
    ssr[...] = scr[...]

    flat_big = (lax.broadcasted_iota(jnp.int32, (_ROWS, 128), 0) * 128
                + lax.broadcasted_iota(jnp.int32, (_ROWS, 128), 1))
    flat_sm = (lax.broadcasted_iota(jnp.int32, (_PRE_ROWS, 128), 0) * 128
               + lax.broadcasted_iota(jnp.int32, (_PRE_ROWS, 128), 1))
    flat_out = (lax.broadcasted_iota(jnp.int32, (_OUT_ROWS, 128), 0) * 128
                + lax.broadcasted_iota(jnp.int32, (_OUT_ROWS, 128), 1))

    zeros_sm = jnp.zeros((_PRE_ROWS, 128), jnp.float32)

    def extract(s):
        m = jnp.max(s)
        sel = jnp.min(jnp.where(s == m, flat_big, _NPAD))
        oh = flat_big == sel
        gx1 = jnp.sum(jnp.where(oh, x1s[...], 0.0))
        gy1 = jnp.sum(jnp.where(oh, y1s[...], 0.0))
        gx2 = jnp.sum(jnp.where(oh, x2s[...], 0.0))
        gy2 = jnp.sum(jnp.where(oh, y2s[...], 0.0))
        return oh, gx1, gy1, gx2, gy2

    def nms_step(box, carry):
        gx1, gy1, gx2, gy2 = box
        kx1, ky1, kx2, ky2, ka, sx1, sy1, sx2, sy2, kj, sj = carry
        aj = (gx2 - gx1 + 1.0) * (gy2 - gy1 + 1.0)
        xx1 = jnp.maximum(gx1, kx1)
        yy1 = jnp.maximum(gy1, ky1)
        xx2 = jnp.minimum(gx2, kx2)
        yy2 = jnp.minimum(gy2, ky2)
        w = jnp.maximum(xx2 - xx1 + 1.0, 0.0)
        h = jnp.maximum(yy2 - yy1 + 1.0, 0.0)
        inter = w * h
        iou = inter / (aj + ka - inter)
        viol = jnp.max(jnp.where(iou > _NMS_T, 1.0, 0.0))
        keptj = viol < 0.5
        tgtk = (flat_sm == kj) & keptj
        kx1 = jnp.where(tgtk, gx1, kx1)
        ky1 = jnp.where(tgtk, gy1, ky1)
        kx2 = jnp.where(tgtk, gx2, kx2)
        ky2 = jnp.where(tgtk, gy2, ky2)
        ka = jnp.where(tgtk, aj, ka)
        tgts = (flat_sm == sj) & (~keptj)
        sx1 = jnp.where(tgts, gx1, sx1)
        sy1 = jnp.where(tgts, gy1, sy1)
        sx2 = jnp.where(tgts, gx2, sx2)
        sy2 = jnp.where(tgts, gy2, sy2)
        one = jnp.where(keptj, 1, 0)
        return (kx1, ky1, kx2, ky2, ka, sx1, sy1, sx2, sy2,
                kj + one, sj + (1 - one))

    def body(j, carry):
        s = ssr[...]
        oh1, *box1 = extract(s)
        s2 = jnp.where(oh1, _NEG, s)
        oh2, *box2 = extract(s2)
        ssr[...] = jnp.where(oh2, _NEG, s2)
        carry = nms_step(box1, carry)
        return nms_step(box2, carry)

    init = (zeros_sm, zeros_sm,
            jnp.full((_PRE_ROWS, 128), -2.0, jnp.float32),
            jnp.full((_PRE_ROWS, 128), -2.0, jnp.float32),
            jnp.full((_PRE_ROWS, 128), 1.0, jnp.float32),
            zeros_sm, zeros_sm, zeros_sm, zeros_sm,
            jnp.int32(0), jnp.int32(0))
    (kx1, ky1, kx2, ky2, _ka,
     sx1, sy1, sx2, sy2, kj, _sj) = lax.fori_loop(0, _PRE // 2, body, init)

    o1r[...] = kx1[0:_OUT_ROWS, :]
    o2r[...] = ky1[0:_OUT_ROWS, :]
    o3r[...] = kx2[0:_OUT_ROWS, :]
    o4r[...] = ky2[0:_OUT_ROWS, :]

    nfill = jnp.maximum(0, _POST - kj)

    def fill(t, _):
        ohs = flat_sm == t
        f1 = jnp.sum(jnp.where(ohs, sx1, 0.0))
        f2 = jnp.sum(jnp.where(ohs, sy1, 0.0))
        f3 = jnp.sum(jnp.where(ohs, sx2, 0.0))
        f4 = jnp.sum(jnp.where(ohs, sy2, 0.0))
        tgt = flat_out == (kj + t)
        o1r[...] = jnp.where(tgt, f1, o1r[...])
        o2r[...] = jnp.where(tgt, f2, o2r[...])
        o3r[...] = jnp.where(tgt, f3, o3r[...])
        o4r[...] = jnp.where(tgt, f4, o4r[...])
        return 0

    lax.fori_loop(0, nfill, fill, 0)


@jax.jit
def kernel(anchors, objectness, pred_bbox_deltas):
    pad = _NPAD - _N
    s = jnp.pad(objectness.reshape(-1), (0, pad),
                constant_values=_NEG).reshape(_ROWS, 128)
    a = jnp.pad(anchors, ((0, pad), (0, 0)))
    d = jnp.pad(pred_bbox_deltas, ((0, pad), (0, 0)))
    cols = [a[:, i].reshape(_ROWS, 128) for i in range(4)]
    dcols = [d[:, i].reshape(_ROWS, 128) for i in range(4)]
    out_shape = [jax.ShapeDtypeStruct((_OUT_ROWS, 128), jnp.float32)] * 4
    scratch = [pltpu.VMEM((_ROWS, 128), jnp.float32)] * 5
    o1, o2, o3, o4 = pl.pallas_call(
        _rpn_kernel,
        out_shape=out_shape,
        scratch_shapes=scratch,
    )(*cols, *dcols, s)
    return jnp.stack([o.reshape(-1)[:_POST] for o in (o1, o2, o3, o4)],
                     axis=1)


# speedup vs baseline: 2.0054x; 1.0029x over previous
import jax
import jax.numpy as jnp
import numpy as np
from jax import lax
from jax.experimental import pallas as pl
from jax.experimental.pallas import tpu as pltpu

_N = 20000
_ROWS = 160
_NPAD = _ROWS * 128
_PRE = 2000
_PRE_ROWS = 16
_POST = 1000
_OUT_ROWS = 8
_NMS_T = 0.7
_IMG_W = 1024.0
_IMG_H = 1024.0
_BBOX_CLIP = float(np.log(1000.0 / 16.0))
_NEG = float("-inf")


def _rpn_kernel(a0r, a1r, a2r, a3r, d0r, d1r, d2r, d3r, scr,
                o1r, o2r, o3r, o4r,
                x1s, y1s, x2s, y2s, ssr):
    a0 = a0r[...]
    a1 = a1r[...]
    a2 = a2r[...]
    a3 = a3r[...]
    widths = a2 - a0 + 1.0
    heights = a3 - a1 + 1.0
    ctr_x = a0 + 0.5 * widths
    ctr_y = a1 + 0.5 * heights
    dw = jnp.minimum(d2r[...], _BBOX_CLIP)
    dh = jnp.minimum(d3r[...], _BBOX_CLIP)
    pcx = d0r[...] * widths + ctr_x
    pcy = d1r[...] * heights + ctr_y
    pw = jnp.exp(dw) * widths
    ph = jnp.exp(dh) * heights
    x1s[...] = jnp.clip(pcx - 0.5 * pw, 0.0, _IMG_W - 1.0)
    y1s[...] = jnp.clip(pcy - 0.5 * ph, 0.0, _IMG_H - 1.0)
    x2s[...] = jnp.clip(pcx + 0.5 * pw - 1.0, 0.0, _IMG_W - 1.0)
    y2s[...] = jnp.clip(pcy + 0.5 * ph - 1.0, 0.0, _IMG_H - 1.0)
    ssr[...] = scr[...]

    flat_big = (lax.broadcasted_iota(jnp.int32, (_ROWS, 128), 0) * 128
                + lax.broadcasted_iota(jnp.int32, (_ROWS, 128), 1))
    flat_sm = (lax.broadcasted_iota(jnp.int32, (_PRE_ROWS, 128), 0) * 128
               + lax.broadcasted_iota(jnp.int32, (_PRE_ROWS, 128), 1))
    flat_out = (lax.broadcasted_iota(jnp.int32, (_OUT_ROWS, 128), 0) * 128
                + lax.broadcasted_iota(jnp.int32, (_OUT_ROWS, 128), 1))

    zeros_sm = jnp.zeros((_PRE_ROWS, 128), jnp.float32)

    def extract(s):
        m = jnp.max(s)
        sel = jnp.min(jnp.where(s == m, flat_big, _NPAD))
        oh = flat_big == sel
        gx1 = jnp.sum(jnp.where(oh, x1s[...], 0.0))
        gy1 = jnp.sum(jnp.where(oh, y1s[...], 0.0))
        gx2 = jnp.sum(jnp.where(oh, x2s[...], 0.0))
        gy2 = jnp.sum(jnp.where(oh, y2s[...], 0.0))
        return oh, gx1, gy1, gx2, gy2

    def nms_step(box, carry):
        gx1, gy1, gx2, gy2 = box
        kx1, ky1, kx2, ky2, ka, sx1, sy1, sx2, sy2, kj, sj = carry
        aj = (gx2 - gx1 + 1.0) * (gy2 - gy1 + 1.0)
        xx1 = jnp.maximum(gx1, kx1)
        yy1 = jnp.maximum(gy1, ky1)
        xx2 = jnp.minimum(gx2, kx2)
        yy2 = jnp.minimum(gy2, ky2)
        w = jnp.maximum(xx2 - xx1 + 1.0, 0.0)
        h = jnp.maximum(yy2 - yy1 + 1.0, 0.0)
        inter = w * h
        iou = inter / (aj + ka - inter)
        viol = jnp.max(jnp.where(iou > _NMS_T, 1.0, 0.0))
        keptj = viol < 0.5
        tgtk = (flat_sm == kj) & keptj
        kx1 = jnp.where(tgtk, gx1, kx1)
        ky1 = jnp.where(tgtk, gy1, ky1)
        kx2 = jnp.where(tgtk, gx2, kx2)
        ky2 = jnp.where(tgtk, gy2, ky2)
        ka = jnp.where(tgtk, aj, ka)
        tgts = (flat_sm == sj) & (~keptj)
        sx1 = jnp.where(tgts, gx1, sx1)
        sy1 = jnp.where(tgts, gy1, sy1)
        sx2 = jnp.where(tgts, gx2, sx2)
        sy2 = jnp.where(tgts, gy2, sy2)
        one = jnp.where(keptj, 1, 0)
        return (kx1, ky1, kx2, ky2, ka, sx1, sy1, sx2, sy2,
                kj + one, sj + (1 - one))

    def body(j, carry):
        s = ssr[...]
        oh1, *box1 = extract(s)
        s2 = jnp.where(oh1, _NEG, s)
        oh2, *box2 = extract(s2)
        s3 = jnp.where(oh2, _NEG, s2)
        oh3, *box3 = extract(s3)
        s4 = jnp.where(oh3, _NEG, s3)
        oh4, *box4 = extract(s4)
        ssr[...] = jnp.where(oh4, _NEG, s4)
        carry = nms_step(box1, carry)
        carry = nms_step(box2, carry)
        carry = nms_step(box3, carry)
        return nms_step(box4, carry)

    init = (zeros_sm, zeros_sm,
            jnp.full((_PRE_ROWS, 128), -2.0, jnp.float32),
            jnp.full((_PRE_ROWS, 128), -2.0, jnp.float32),
            jnp.full((_PRE_ROWS, 128), 1.0, jnp.float32),
            zeros_sm, zeros_sm, zeros_sm, zeros_sm,
            jnp.int32(0), jnp.int32(0))
    (kx1, ky1, kx2, ky2, _ka,
     sx1, sy1, sx2, sy2, kj, _sj) = lax.fori_loop(0, _PRE // 4, body, init)

    o1r[...] = kx1[0:_OUT_ROWS, :]
    o2r[...] = ky1[0:_OUT_ROWS, :]
    o3r[...] = kx2[0:_OUT_ROWS, :]
    o4r[...] = ky2[0:_OUT_ROWS, :]

    nfill = jnp.maximum(0, _POST - kj)

    def fill(t, _):
        ohs = flat_sm == t
        f1 = jnp.sum(jnp.where(ohs, sx1, 0.0))
        f2 = jnp.sum(jnp.where(ohs, sy1, 0.0))
        f3 = jnp.sum(jnp.where(ohs, sx2, 0.0))
        f4 = jnp.sum(jnp.where(ohs, sy2, 0.0))
        tgt = flat_out == (kj + t)
        o1r[...] = jnp.where(tgt, f1, o1r[...])
        o2r[...] = jnp.where(tgt, f2, o2r[...])
        o3r[...] = jnp.where(tgt, f3, o3r[...])
        o4r[...] = jnp.where(tgt, f4, o4r[...])
        return 0

    lax.fori_loop(0, nfill, fill, 0)


@jax.jit
def kernel(anchors, objectness, pred_bbox_deltas):
    pad = _NPAD - _N
    s = jnp.pad(objectness.reshape(-1), (0, pad),
                constant_values=_NEG).reshape(_ROWS, 128)
    a = jnp.pad(anchors, ((0, pad), (0, 0)))
    d = jnp.pad(pred_bbox_deltas, ((0, pad), (0, 0)))
    cols = [a[:, i].reshape(_ROWS, 128) for i in range(4)]
    dcols = [d[:, i].reshape(_ROWS, 128) for i in range(4)]
    out_shape = [jax.ShapeDtypeStruct((_OUT_ROWS, 128), jnp.float32)] * 4
    scratch = [pltpu.VMEM((_ROWS, 128), jnp.float32)] * 5
    o1, o2, o3, o4 = pl.pallas_call(
        _rpn_kernel,
        out_shape=out_shape,
        scratch_shapes=scratch,
    )(*cols, *dcols, s)
    return jnp.stack([o.reshape(-1)[:_POST] for o in (o1, o2, o3, o4)],
                     axis=1)
